# baseline (speedup 1.0000x reference)
import functools

import numpy as np
import jax
import jax.numpy as jnp
from jax import lax
from jax.experimental import pallas as pl
from jax.experimental.pallas import tpu as pltpu
from jax.experimental.pallas import tpu_sc as plsc

NV = 2048
NE = 32768
Q = NV // 2
QW = Q * Q
NS = 16
NC = 2
EPT = NE // NS
CHUNK_EDGES = 64
CH = 2 * CHUNK_EDGES
NCHUNK = EPT // CHUNK_EDGES
TPW = QW // NS

_N = 33
_n = np.arange(_N, dtype=np.float64)
_x = np.cos(np.pi * (_n + 0.5) / _N)
_kern = np.exp(-2.5 * (_x + 1.0))
_c = np.array([(2.0 / _N) * np.sum(np.cos(np.pi * o * (_n + 0.5) / _N) * _kern)
               for o in range(_N)])
COEF_I = float(0.5 * _c[0] + _c[4::4].sum() - _c[2::4].sum())
COEF_M = float(_c[1] + _c[5::4].sum() - _c[3::4].sum())
COEF_P = float(2.0 * (_c[2::4].sum() + _c[3::4].sum()))
SCALE = float(np.sqrt(NV))


def _sc_scatter_body(ei0, ei1, ew, zeros, a_out, degp_out,
                     r_v, c_v, w_v, idx_v, didx_v, val_v, a_sh, deg_sh,
                     sem_a, sem_d):
    cid = lax.axis_index("c")
    sid = lax.axis_index("s")
    base = sid * EPT

    pltpu.sync_copy(ei0.at[pl.ds(base, EPT)], r_v)
    pltpu.sync_copy(ei1.at[pl.ds(base, EPT)], c_v)
    pltpu.sync_copy(ew.at[pl.ds(base, EPT)], w_v)
    pltpu.sync_copy(zeros.at[pl.ds(sid * (NV // NS), NV // NS)],
                    deg_sh.at[pl.ds(sid * (NV // NS), NV // NS)])

    for p in range(2):
        pltpu.sync_copy(zeros.at[pl.ds(sid * TPW, TPW)],
                        a_sh.at[pl.ds(sid * TPW, TPW)])
        plsc.subcore_barrier()

        def chunk_body(j, carry):
            for v in range(CHUNK_EDGES // 16):
                off = j * CHUNK_EDGES + v * 16
                r16 = r_v[pl.ds(off, 16)]
                c16 = c_v[pl.ds(off, 16)]
                w16 = w_v[pl.ds(off, 16)]
                for d in range(2):
                    row = r16 if d == 0 else c16
                    col = c16 if d == 0 else r16
                    inq = ((lax.shift_right_logical(row, 10) == p)
                           & (lax.shift_right_logical(col, 10) == cid))
                    lidx = lax.shift_left(row & (Q - 1), 10) | (col & (Q - 1))
                    val = jnp.where(inq, w16 * 0.5, 0.0)
                    ent = d * CHUNK_EDGES + v * 16
                    idx_v[j, pl.ds(ent, 16)] = lidx
                    didx_v[j, pl.ds(ent, 16)] = row
                    val_v[j, pl.ds(ent, 16)] = val
            pltpu.async_copy(val_v.at[j], a_sh.at[idx_v.at[j]], sem_a, add=True)
            pltpu.async_copy(val_v.at[j], deg_sh.at[didx_v.at[j]], sem_d, add=True)
            return carry

        lax.fori_loop(0, NCHUNK, chunk_body, 0)

        def drain_body(j, carry):
            pltpu.make_async_copy(val_v.at[j], a_sh.at[idx_v.at[j]], sem_a).wait()
            pltpu.make_async_copy(val_v.at[j], deg_sh.at[didx_v.at[j]], sem_d).wait()
            return carry

        lax.fori_loop(0, NCHUNK, drain_body, 0)
        plsc.subcore_barrier()
        pltpu.sync_copy(a_sh.at[pl.ds(sid * TPW, TPW)], a_out.at[p, cid, sid])
        plsc.subcore_barrier()

    @pl.when(sid == 0)
    def _():
        pltpu.sync_copy(deg_sh, degp_out.at[cid])


def _sc_scatter(ei0, ei1, ew, zeros):
    mesh = plsc.VectorSubcoreMesh(core_axis_name="c", subcore_axis_name="s")
    f = pl.kernel(
        _sc_scatter_body,
        out_type=[
            jax.ShapeDtypeStruct((2, NC, NS, TPW), jnp.float32),
            jax.ShapeDtypeStruct((NC, NV), jnp.float32),
        ],
        mesh=mesh,
        scratch_types=[
            pltpu.VMEM((EPT,), jnp.int32),
            pltpu.VMEM((EPT,), jnp.int32),
            pltpu.VMEM((EPT,), jnp.float32),
            pltpu.VMEM((NCHUNK, CH), jnp.int32),
            pltpu.VMEM((NCHUNK, CH), jnp.int32),
            pltpu.VMEM((NCHUNK, CH), jnp.float32),
            pltpu.VMEM_SHARED((QW,), jnp.float32),
            pltpu.VMEM_SHARED((NV,), jnp.float32),
            pltpu.SemaphoreType.DMA,
            pltpu.SemaphoreType.DMA,
        ],
    )
    return f(ei0, ei1, ew, zeros)


BI = BJ = BK = 512
NI = NV // BI
NJ = NV // BJ
NK = NV // BK


def _mm_body(lhs, rhs, aij, degp, out):
    i = pl.program_id(0)
    j = pl.program_id(1)
    k = pl.program_id(2)

    degk = degp[0, pl.ds(k * BK, BK)] + degp[1, pl.ds(k * BK, BK)]
    dinv2 = jnp.where(degk > 0, 1.0 / degk, 0.0)
    a = lhs[0, 0] * dinv2[None, :]
    b = rhs[0, 0]
    ahi = a.astype(jnp.bfloat16)
    alo = (a - ahi.astype(jnp.float32)).astype(jnp.bfloat16)
    bhi = b.astype(jnp.bfloat16)
    blo = (b - bhi.astype(jnp.float32)).astype(jnp.bfloat16)
    part = (jnp.dot(ahi, bhi, preferred_element_type=jnp.float32)
            + jnp.dot(ahi, blo, preferred_element_type=jnp.float32)
            + jnp.dot(alo, bhi, preferred_element_type=jnp.float32))

    @pl.when(k == 0)
    def _():
        out[...] = jnp.zeros((BI, BJ), jnp.float32)

    out[...] += part

    @pl.when(k == NK - 1)
    def _():
        degi = degp[0, pl.ds(i * BI, BI)] + degp[1, pl.ds(i * BI, BI)]
        degj = degp[0, pl.ds(j * BJ, BJ)] + degp[1, pl.ds(j * BJ, BJ)]
        dinvi = jnp.where(degi > 0, lax.rsqrt(degi), 0.0)
        dinvj = jnp.where(degj > 0, lax.rsqrt(degj), 0.0)
        rows = i * BI + lax.broadcasted_iota(jnp.int32, (BI, BJ), 0)
        cols = j * BJ + lax.broadcasted_iota(jnp.int32, (BI, BJ), 1)
        eye = (rows == cols).astype(jnp.float32)
        dd = dinvi[:, None] * dinvj[None, :]
        out[...] = SCALE * (dd * (COEF_P * out[...] - COEF_M * aij[0, 0])
                            + COEF_I * eye)


def _mm_call(asym4, degp):
    return pl.pallas_call(
        _mm_body,
        grid=(NI, NJ, NK),
        in_specs=[
            pl.BlockSpec((1, 1, BI, BK), lambda i, j, k: (i // 2, k // 2, i % 2, k % 2)),
            pl.BlockSpec((1, 1, BK, BJ), lambda i, j, k: (k // 2, j // 2, k % 2, j % 2)),
            pl.BlockSpec((1, 1, BI, BJ), lambda i, j, k: (i // 2, j // 2, i % 2, j % 2)),
            pl.BlockSpec((NC, NV), lambda i, j, k: (0, 0)),
        ],
        out_specs=pl.BlockSpec((BI, BJ), lambda i, j, k: (i, j)),
        out_shape=jax.ShapeDtypeStruct((NV, NV), jnp.float32),
        compiler_params=pltpu.CompilerParams(
            dimension_semantics=("parallel", "parallel", "arbitrary")),
    )(asym4, asym4, asym4, degp)


@jax.jit
def kernel(edge_index, edge_weight):
    zeros = jnp.zeros((QW,), jnp.float32)
    a4, degp = _sc_scatter(edge_index[0], edge_index[1], edge_weight, zeros)
    asym4 = a4.reshape(2, NC, Q, Q)
    out = _mm_call(asym4, degp)
    return out.reshape(1, NV, NV)


# speedup vs baseline: 1.3808x; 1.3808x over previous
import functools

import numpy as np
import jax
import jax.numpy as jnp
from jax import lax
from jax.experimental import pallas as pl
from jax.experimental.pallas import tpu as pltpu
from jax.experimental.pallas import tpu_sc as plsc

NV = 2048
NE = 32768
Q = NV // 2
QW = Q * Q
NS = 16
NC = 2
EPT = NE // NS
CHUNK_EDGES = 64
CH = 2 * CHUNK_EDGES
NCHUNK = EPT // CHUNK_EDGES
TPW = QW // NS

_N = 33
_n = np.arange(_N, dtype=np.float64)
_x = np.cos(np.pi * (_n + 0.5) / _N)
_kern = np.exp(-2.5 * (_x + 1.0))
_c = np.array([(2.0 / _N) * np.sum(np.cos(np.pi * o * (_n + 0.5) / _N) * _kern)
               for o in range(_N)])
COEF_I = float(0.5 * _c[0] + _c[4::4].sum() - _c[2::4].sum())
COEF_M = float(_c[1] + _c[5::4].sum() - _c[3::4].sum())
COEF_P = float(2.0 * (_c[2::4].sum() + _c[3::4].sum()))
SCALE = float(np.sqrt(NV))


def _sc_scatter_body(ei0, ei1, ew, zeros, a_out, degp_out,
                     r_v, c_v, w_v, idx_v, didx_v, val_v, a_sh, deg_sh,
                     sem_a, sem_d):
    cid = lax.axis_index("c")
    sid = lax.axis_index("s")
    base = sid * EPT

    pltpu.sync_copy(ei0.at[pl.ds(base, EPT)], r_v)
    pltpu.sync_copy(ei1.at[pl.ds(base, EPT)], c_v)
    pltpu.sync_copy(ew.at[pl.ds(base, EPT)], w_v)
    pltpu.sync_copy(zeros.at[pl.ds(sid * (NV // NS), NV // NS)],
                    deg_sh.at[pl.ds(sid * (NV // NS), NV // NS)])

    for p in range(2):
        pltpu.sync_copy(zeros.at[pl.ds(sid * TPW, TPW)],
                        a_sh.at[pl.ds(sid * TPW, TPW)])
        plsc.subcore_barrier()

        def chunk_body(j, carry):
            for v in range(CHUNK_EDGES // 16):
                off = j * CHUNK_EDGES + v * 16
                r16 = r_v[pl.ds(off, 16)]
                c16 = c_v[pl.ds(off, 16)]
                w16 = w_v[pl.ds(off, 16)]
                for d in range(2):
                    row = r16 if d == 0 else c16
                    col = c16 if d == 0 else r16
                    inq = ((lax.shift_right_logical(row, 10) == p)
                           & (lax.shift_right_logical(col, 10) == cid))
                    lidx = lax.shift_left(row & (Q - 1), 10) | (col & (Q - 1))
                    val = jnp.where(inq, w16 * 0.5, 0.0)
                    ent = d * CHUNK_EDGES + v * 16
                    idx_v[j, pl.ds(ent, 16)] = lidx
                    didx_v[j, pl.ds(ent, 16)] = row
                    val_v[j, pl.ds(ent, 16)] = val
            pltpu.async_copy(val_v.at[j], a_sh.at[idx_v.at[j]], sem_a, add=True)
            pltpu.async_copy(val_v.at[j], deg_sh.at[didx_v.at[j]], sem_d, add=True)
            return carry

        lax.fori_loop(0, NCHUNK, chunk_body, 0)

        def drain_body(j, carry):
            pltpu.make_async_copy(val_v.at[j], a_sh.at[idx_v.at[j]], sem_a).wait()
            pltpu.make_async_copy(val_v.at[j], deg_sh.at[didx_v.at[j]], sem_d).wait()
            return carry

        lax.fori_loop(0, NCHUNK, drain_body, 0)
        plsc.subcore_barrier()
        pltpu.sync_copy(a_sh.at[pl.ds(sid * TPW, TPW)], a_out.at[p, cid, sid])
        plsc.subcore_barrier()

    @pl.when(sid == 0)
    def _():
        pltpu.sync_copy(deg_sh, degp_out.at[cid])


def _sc_scatter(ei0, ei1, ew, zeros):
    mesh = plsc.VectorSubcoreMesh(core_axis_name="c", subcore_axis_name="s")
    f = pl.kernel(
        _sc_scatter_body,
        out_type=[
            jax.ShapeDtypeStruct((2, NC, NS, TPW), jnp.float32),
            jax.ShapeDtypeStruct((NC, NV), jnp.float32),
        ],
        mesh=mesh,
        scratch_types=[
            pltpu.VMEM((EPT,), jnp.int32),
            pltpu.VMEM((EPT,), jnp.int32),
            pltpu.VMEM((EPT,), jnp.float32),
            pltpu.VMEM((NCHUNK, CH), jnp.int32),
            pltpu.VMEM((NCHUNK, CH), jnp.int32),
            pltpu.VMEM((NCHUNK, CH), jnp.float32),
            pltpu.VMEM_SHARED((QW,), jnp.float32),
            pltpu.VMEM_SHARED((NV,), jnp.float32),
            pltpu.SemaphoreType.DMA,
            pltpu.SemaphoreType.DMA,
        ],
    )
    return f(ei0, ei1, ew, zeros)


BI = BJ = 512
NI = NV // BI
NJ = NV // BJ


def _prescale_body(a, degp, out):
    qc = pl.program_id(1)
    degk = degp[0, pl.ds(qc * Q, Q)] + degp[1, pl.ds(qc * Q, Q)]
    dinv2 = jnp.where(degk > 0, 1.0 / degk, 0.0)
    out[0, 0] = a[0, 0] * dinv2[None, :]


def _prescale_call(asym4, degp):
    return pl.pallas_call(
        _prescale_body,
        grid=(2, 2),
        in_specs=[
            pl.BlockSpec((1, 1, Q, Q), lambda qr, qc: (qr, qc, 0, 0)),
            pl.BlockSpec((NC, NV), lambda qr, qc: (0, 0)),
        ],
        out_specs=pl.BlockSpec((1, 1, Q, Q), lambda qr, qc: (qr, qc, 0, 0)),
        out_shape=jax.ShapeDtypeStruct((2, 2, Q, Q), jnp.float32),
        compiler_params=pltpu.CompilerParams(
            dimension_semantics=("parallel", "parallel")),
    )(asym4, degp)


def _mm_body(lhs, rhs, aij, degp, out):
    i = pl.program_id(0)
    j = pl.program_id(1)

    acc = (jnp.dot(lhs[0, 0], rhs[0, 0], preferred_element_type=jnp.float32)
           + jnp.dot(lhs[0, 1], rhs[1, 0], preferred_element_type=jnp.float32))

    degi = degp[0, pl.ds(i * BI, BI)] + degp[1, pl.ds(i * BI, BI)]
    degj = degp[0, pl.ds(j * BJ, BJ)] + degp[1, pl.ds(j * BJ, BJ)]
    dinvi = jnp.where(degi > 0, lax.rsqrt(degi), 0.0)
    dinvj = jnp.where(degj > 0, lax.rsqrt(degj), 0.0)
    rows = i * BI + lax.broadcasted_iota(jnp.int32, (BI, BJ), 0)
    cols = j * BJ + lax.broadcasted_iota(jnp.int32, (BI, BJ), 1)
    eye = (rows == cols).astype(jnp.float32)
    dd = dinvi[:, None] * dinvj[None, :]
    out[...] = SCALE * (dd * (COEF_P * acc - COEF_M * aij[0, 0])
                        + COEF_I * eye)


def _mm_call(ascaled, asym4, degp):
    return pl.pallas_call(
        _mm_body,
        grid=(NI, NJ),
        in_specs=[
            pl.BlockSpec((1, 2, BI, Q), lambda i, j: (i // 2, 0, i % 2, 0)),
            pl.BlockSpec((2, 1, Q, BJ), lambda i, j: (0, j // 2, 0, j % 2)),
            pl.BlockSpec((1, 1, BI, BJ), lambda i, j: (i // 2, j // 2, i % 2, j % 2)),
            pl.BlockSpec((NC, NV), lambda i, j: (0, 0)),
        ],
        out_specs=pl.BlockSpec((BI, BJ), lambda i, j: (i, j)),
        out_shape=jax.ShapeDtypeStruct((NV, NV), jnp.float32),
        compiler_params=pltpu.CompilerParams(
            dimension_semantics=("parallel", "parallel")),
    )(ascaled, asym4, asym4, degp)


@jax.jit
def kernel(edge_index, edge_weight):
    zeros = jnp.zeros((QW,), jnp.float32)
    a4, degp = _sc_scatter(edge_index[0], edge_index[1], edge_weight, zeros)
    asym4 = a4.reshape(2, NC, Q, Q)
    ascaled = _prescale_call(asym4, degp)
    out = _mm_call(ascaled, asym4, degp)
    return out.reshape(1, NV, NV)


# speedup vs baseline: 1.5749x; 1.1405x over previous
import functools

import numpy as np
import jax
import jax.numpy as jnp
from jax import lax
from jax.experimental import pallas as pl
from jax.experimental.pallas import tpu as pltpu
from jax.experimental.pallas import tpu_sc as plsc

NV = 2048
NE = 32768
Q = NV // 2
QW = Q * Q
NS = 16
NC = 2
EPT = NE // NS
CHUNK_EDGES = 64
CH = 2 * CHUNK_EDGES
NCHUNK = EPT // CHUNK_EDGES
TPW = QW // NS

_N = 33
_n = np.arange(_N, dtype=np.float64)
_x = np.cos(np.pi * (_n + 0.5) / _N)
_kern = np.exp(-2.5 * (_x + 1.0))
_c = np.array([(2.0 / _N) * np.sum(np.cos(np.pi * o * (_n + 0.5) / _N) * _kern)
               for o in range(_N)])
COEF_I = float(0.5 * _c[0] + _c[4::4].sum() - _c[2::4].sum())
COEF_M = float(_c[1] + _c[5::4].sum() - _c[3::4].sum())
COEF_P = float(2.0 * (_c[2::4].sum() + _c[3::4].sum()))
SCALE = float(np.sqrt(NV))


def _sc_scatter_body(ei0, ei1, ew, zeros, a_out, degp_out,
                     r_v, c_v, w_v, idx_v, didx_v, val_v, a_sh, deg_sh,
                     sem_a, sem_d):
    cid = lax.axis_index("c")
    sid = lax.axis_index("s")
    base = sid * EPT

    pltpu.sync_copy(ei0.at[pl.ds(base, EPT)], r_v)
    pltpu.sync_copy(ei1.at[pl.ds(base, EPT)], c_v)
    pltpu.sync_copy(ew.at[pl.ds(base, EPT)], w_v)
    pltpu.sync_copy(zeros.at[pl.ds(sid * (NV // NS), NV // NS)],
                    deg_sh.at[pl.ds(sid * (NV // NS), NV // NS)])

    for p in range(2):
        pltpu.sync_copy(zeros, a_sh.at[pl.ds(sid * TPW, TPW)])
        plsc.subcore_barrier()

        def chunk_body(j, carry):
            for v in range(CHUNK_EDGES // 16):
                off = j * CHUNK_EDGES + v * 16
                r16 = r_v[pl.ds(off, 16)]
                c16 = c_v[pl.ds(off, 16)]
                w16 = w_v[pl.ds(off, 16)]
                for d in range(2):
                    row = r16 if d == 0 else c16
                    col = c16 if d == 0 else r16
                    inq = ((lax.shift_right_logical(row, 10) == p)
                           & (lax.shift_right_logical(col, 10) == cid))
                    lidx = lax.shift_left(row & (Q - 1), 10) | (col & (Q - 1))
                    val = jnp.where(inq, w16 * 0.5, 0.0)
                    ent = d * CHUNK_EDGES + v * 16
                    idx_v[j, pl.ds(ent, 16)] = lidx
                    didx_v[j, pl.ds(ent, 16)] = row
                    val_v[j, pl.ds(ent, 16)] = val
            pltpu.async_copy(val_v.at[j], a_sh.at[idx_v.at[j]], sem_a, add=True)
            pltpu.async_copy(val_v.at[j], deg_sh.at[didx_v.at[j]], sem_d, add=True)
            return carry

        lax.fori_loop(0, NCHUNK, chunk_body, 0)

        def drain_body(j, carry):
            pltpu.make_async_copy(val_v.at[j], a_sh.at[idx_v.at[j]], sem_a).wait()
            pltpu.make_async_copy(val_v.at[j], deg_sh.at[didx_v.at[j]], sem_d).wait()
            return carry

        lax.fori_loop(0, NCHUNK, drain_body, 0)
        plsc.subcore_barrier()
        pltpu.sync_copy(a_sh.at[pl.ds(sid * TPW, TPW)], a_out.at[p, cid, sid])
        plsc.subcore_barrier()

    @pl.when(sid == 0)
    def _():
        pltpu.sync_copy(deg_sh, degp_out.at[cid])


def _sc_scatter(ei0, ei1, ew, zeros):
    mesh = plsc.VectorSubcoreMesh(core_axis_name="c", subcore_axis_name="s")
    f = pl.kernel(
        _sc_scatter_body,
        out_type=[
            jax.ShapeDtypeStruct((2, NC, NS, TPW), jnp.float32),
            jax.ShapeDtypeStruct((NC, NV), jnp.float32),
        ],
        mesh=mesh,
        scratch_types=[
            pltpu.VMEM((EPT,), jnp.int32),
            pltpu.VMEM((EPT,), jnp.int32),
            pltpu.VMEM((EPT,), jnp.float32),
            pltpu.VMEM((NCHUNK, CH), jnp.int32),
            pltpu.VMEM((NCHUNK, CH), jnp.int32),
            pltpu.VMEM((NCHUNK, CH), jnp.float32),
            pltpu.VMEM_SHARED((QW,), jnp.float32),
            pltpu.VMEM_SHARED((NV,), jnp.float32),
            pltpu.SemaphoreType.DMA,
            pltpu.SemaphoreType.DMA,
        ],
    )
    return f(ei0, ei1, ew, zeros)


BI = BJ = 512
NI = NV // BI
NJ = NV // BJ


def _prescale_body(a, degp, out):
    qc = pl.program_id(1)
    degk = degp[0, pl.ds(qc * Q, Q)] + degp[1, pl.ds(qc * Q, Q)]
    dinv2 = jnp.where(degk > 0, 1.0 / degk, 0.0)
    rows = TPW // Q
    for s in range(NS):
        out[0, 0, pl.ds(s * rows, rows), :] = (
            a[0, 0, s].reshape(rows, Q) * dinv2[None, :])


def _prescale_call(a4, degp):
    return pl.pallas_call(
        _prescale_body,
        grid=(2, 2),
        in_specs=[
            pl.BlockSpec((1, 1, NS, TPW), lambda qr, qc: (qr, qc, 0, 0)),
            pl.BlockSpec((NC, NV), lambda qr, qc: (0, 0)),
        ],
        out_specs=pl.BlockSpec((1, 1, Q, Q), lambda qr, qc: (qr, qc, 0, 0)),
        out_shape=jax.ShapeDtypeStruct((2, 2, Q, Q), jnp.float32),
        compiler_params=pltpu.CompilerParams(
            dimension_semantics=("parallel", "parallel")),
    )(a4, degp)


def _mm_body(lhs, rhs, aij, degp, out):
    i = pl.program_id(0)
    j = pl.program_id(1)

    acc = (jnp.dot(lhs[0, 0], rhs[0, 0], preferred_element_type=jnp.float32)
           + jnp.dot(lhs[0, 1], rhs[1, 0], preferred_element_type=jnp.float32))

    degi = degp[0, pl.ds(i * BI, BI)] + degp[1, pl.ds(i * BI, BI)]
    degj = degp[0, pl.ds(j * BJ, BJ)] + degp[1, pl.ds(j * BJ, BJ)]
    dinvi = jnp.where(degi > 0, lax.rsqrt(degi), 0.0)
    sdj = jnp.sqrt(degj)
    rows = i * BI + lax.broadcasted_iota(jnp.int32, (BI, BJ), 0)
    cols = j * BJ + lax.broadcasted_iota(jnp.int32, (BI, BJ), 1)
    eye = (rows == cols).astype(jnp.float32)
    dd = dinvi[:, None] * sdj[None, :]
    out[...] = SCALE * (dd * (COEF_P * acc - COEF_M * aij[0, 0])
                        + COEF_I * eye)


def _mm_call(ascaled, degp):
    return pl.pallas_call(
        _mm_body,
        grid=(NI, NJ),
        in_specs=[
            pl.BlockSpec((1, 2, BI, Q), lambda i, j: (i // 2, 0, i % 2, 0)),
            pl.BlockSpec((2, 1, Q, BJ), lambda i, j: (0, j // 2, 0, j % 2)),
            pl.BlockSpec((1, 1, BI, BJ), lambda i, j: (i // 2, j // 2, i % 2, j % 2)),
            pl.BlockSpec((NC, NV), lambda i, j: (0, 0)),
        ],
        out_specs=pl.BlockSpec((BI, BJ), lambda i, j: (i, j)),
        out_shape=jax.ShapeDtypeStruct((NV, NV), jnp.float32),
        compiler_params=pltpu.CompilerParams(
            dimension_semantics=("parallel", "parallel")),
    )(ascaled, ascaled, ascaled, degp)


@jax.jit
def kernel(edge_index, edge_weight):
    zeros = jnp.zeros((TPW,), jnp.float32)
    a4, degp = _sc_scatter(edge_index[0], edge_index[1], edge_weight, zeros)
    ascaled = _prescale_call(a4, degp)
    out = _mm_call(ascaled, degp)
    return out.reshape(1, NV, NV)


# speedup vs baseline: 1.6548x; 1.0507x over previous
import functools

import numpy as np
import jax
import jax.numpy as jnp
from jax import lax
from jax.experimental import pallas as pl
from jax.experimental.pallas import tpu as pltpu
from jax.experimental.pallas import tpu_sc as plsc

NV = 2048
NE = 32768
Q = NV // 2
QW = Q * Q
NS = 16
NC = 2
EPT = NE // NS
CHUNK_EDGES = 64
CH = 2 * CHUNK_EDGES
NCHUNK = EPT // CHUNK_EDGES
TPW = QW // NS

_N = 33
_n = np.arange(_N, dtype=np.float64)
_x = np.cos(np.pi * (_n + 0.5) / _N)
_kern = np.exp(-2.5 * (_x + 1.0))
_c = np.array([(2.0 / _N) * np.sum(np.cos(np.pi * o * (_n + 0.5) / _N) * _kern)
               for o in range(_N)])
COEF_I = float(0.5 * _c[0] + _c[4::4].sum() - _c[2::4].sum())
COEF_M = float(_c[1] + _c[5::4].sum() - _c[3::4].sum())
COEF_P = float(2.0 * (_c[2::4].sum() + _c[3::4].sum()))
SCALE = float(np.sqrt(NV))


def _sc_scatter_body(ei0, ei1, ew, zeros, a_out, degp_out,
                     r_v, c_v, w_v, idx_v, didx_v, val_v, a_sh, deg_sh,
                     sem_a, sem_d):
    cid = lax.axis_index("c")
    sid = lax.axis_index("s")
    base = sid * EPT

    pltpu.sync_copy(ei0.at[pl.ds(base, EPT)], r_v)
    pltpu.sync_copy(ei1.at[pl.ds(base, EPT)], c_v)
    pltpu.sync_copy(ew.at[pl.ds(base, EPT)], w_v)
    pltpu.sync_copy(zeros.at[pl.ds(sid * (NV // NS), NV // NS)],
                    deg_sh.at[pl.ds(sid * (NV // NS), NV // NS)])

    for p in range(2):
        pltpu.sync_copy(zeros, a_sh.at[pl.ds(sid * TPW, TPW)])
        plsc.subcore_barrier()

        def chunk_body(j, carry):
            for v in range(CHUNK_EDGES // 16):
                off = j * CHUNK_EDGES + v * 16
                r16 = r_v[pl.ds(off, 16)]
                c16 = c_v[pl.ds(off, 16)]
                w16 = w_v[pl.ds(off, 16)]
                for d in range(2):
                    row = r16 if d == 0 else c16
                    col = c16 if d == 0 else r16
                    inq = ((lax.shift_right_logical(row, 10) == p)
                           & (lax.shift_right_logical(col, 10) == cid))
                    lidx = lax.shift_left(row & (Q - 1), 10) | (col & (Q - 1))
                    val = jnp.where(inq, w16 * 0.5, 0.0)
                    ent = d * CHUNK_EDGES + v * 16
                    idx_v[j, pl.ds(ent, 16)] = lidx
                    didx_v[j, pl.ds(ent, 16)] = row
                    val_v[j, pl.ds(ent, 16)] = val
            pltpu.async_copy(val_v.at[j], a_sh.at[idx_v.at[j]], sem_a, add=True)
            pltpu.async_copy(val_v.at[j], deg_sh.at[didx_v.at[j]], sem_d, add=True)
            return carry

        lax.fori_loop(0, NCHUNK, chunk_body, 0)

        def drain_body(j, carry):
            pltpu.make_async_copy(val_v.at[j], a_sh.at[idx_v.at[j]], sem_a).wait()
            pltpu.make_async_copy(val_v.at[j], deg_sh.at[didx_v.at[j]], sem_d).wait()
            return carry

        lax.fori_loop(0, NCHUNK, drain_body, 0)
        plsc.subcore_barrier()
        pltpu.sync_copy(a_sh.at[pl.ds(sid * TPW, TPW)], a_out.at[p, cid, sid])
        plsc.subcore_barrier()

    @pl.when(sid == 0)
    def _():
        pltpu.sync_copy(deg_sh, degp_out.at[cid])


def _sc_scatter(ei0, ei1, ew, zeros):
    mesh = plsc.VectorSubcoreMesh(core_axis_name="c", subcore_axis_name="s")
    f = pl.kernel(
        _sc_scatter_body,
        out_type=[
            jax.ShapeDtypeStruct((2, NC, NS, TPW), jnp.float32),
            jax.ShapeDtypeStruct((NC, NV), jnp.float32),
        ],
        mesh=mesh,
        scratch_types=[
            pltpu.VMEM((EPT,), jnp.int32),
            pltpu.VMEM((EPT,), jnp.int32),
            pltpu.VMEM((EPT,), jnp.float32),
            pltpu.VMEM((NCHUNK, CH), jnp.int32),
            pltpu.VMEM((NCHUNK, CH), jnp.int32),
            pltpu.VMEM((NCHUNK, CH), jnp.float32),
            pltpu.VMEM_SHARED((QW,), jnp.float32),
            pltpu.VMEM_SHARED((NV,), jnp.float32),
            pltpu.SemaphoreType.DMA,
            pltpu.SemaphoreType.DMA,
        ],
    )
    return f(ei0, ei1, ew, zeros)


BI = BJ = 512
NI = NV // BI
NJ = NV // BJ


def _prescale_body(a, degp, outf, outb):
    qc = pl.program_id(1)
    degk = degp[0, pl.ds(qc * Q, Q)] + degp[1, pl.ds(qc * Q, Q)]
    dinv2 = jnp.where(degk > 0, 1.0 / degk, 0.0)
    rows = TPW // Q
    for s in range(NS):
        x = a[0, 0, s].reshape(rows, Q) * dinv2[None, :]
        outf[0, 0, pl.ds(s * rows, rows), :] = x
        outb[0, 0, pl.ds(s * rows, rows), :] = x.astype(jnp.bfloat16)


def _prescale_call(a4, degp):
    return pl.pallas_call(
        _prescale_body,
        grid=(2, 2),
        in_specs=[
            pl.BlockSpec((1, 1, NS, TPW), lambda qr, qc: (qr, qc, 0, 0)),
            pl.BlockSpec((NC, NV), lambda qr, qc: (0, 0)),
        ],
        out_specs=[
            pl.BlockSpec((1, 1, Q, Q), lambda qr, qc: (qr, qc, 0, 0)),
            pl.BlockSpec((1, 1, Q, Q), lambda qr, qc: (qr, qc, 0, 0)),
        ],
        out_shape=[
            jax.ShapeDtypeStruct((2, 2, Q, Q), jnp.float32),
            jax.ShapeDtypeStruct((2, 2, Q, Q), jnp.bfloat16),
        ],
        compiler_params=pltpu.CompilerParams(
            dimension_semantics=("parallel", "parallel")),
    )(a4, degp)


def _mm_body(lhs, rhs, aij, degp, out):
    i = pl.program_id(0)
    j = pl.program_id(1)

    acc = (jnp.dot(lhs[0, 0], rhs[0, 0], preferred_element_type=jnp.float32)
           + jnp.dot(lhs[0, 1], rhs[1, 0], preferred_element_type=jnp.float32))

    degi = degp[0, pl.ds(i * BI, BI)] + degp[1, pl.ds(i * BI, BI)]
    degj = degp[0, pl.ds(j * BJ, BJ)] + degp[1, pl.ds(j * BJ, BJ)]
    dinvi = jnp.where(degi > 0, lax.rsqrt(degi), 0.0)
    sdj = jnp.sqrt(degj)
    rows = i * BI + lax.broadcasted_iota(jnp.int32, (BI, BJ), 0)
    cols = j * BJ + lax.broadcasted_iota(jnp.int32, (BI, BJ), 1)
    eye = (rows == cols).astype(jnp.float32)
    dd = dinvi[:, None] * sdj[None, :]
    out[...] = SCALE * (dd * (COEF_P * acc - COEF_M * aij[0, 0])
                        + COEF_I * eye)


def _mm_call(ascf, ascb, degp):
    return pl.pallas_call(
        _mm_body,
        grid=(NI, NJ),
        in_specs=[
            pl.BlockSpec((1, 2, BI, Q), lambda i, j: (i // 2, 0, i % 2, 0)),
            pl.BlockSpec((2, 1, Q, BJ), lambda i, j: (0, j // 2, 0, j % 2)),
            pl.BlockSpec((1, 1, BI, BJ), lambda i, j: (i // 2, j // 2, i % 2, j % 2)),
            pl.BlockSpec((NC, NV), lambda i, j: (0, 0)),
        ],
        out_specs=pl.BlockSpec((BI, BJ), lambda i, j: (i, j)),
        out_shape=jax.ShapeDtypeStruct((NV, NV), jnp.float32),
        compiler_params=pltpu.CompilerParams(
            dimension_semantics=("parallel", "parallel")),
    )(ascb, ascb, ascf, degp)


@jax.jit
def kernel(edge_index, edge_weight):
    zeros = jnp.zeros((TPW,), jnp.float32)
    a4, degp = _sc_scatter(edge_index[0], edge_index[1], edge_weight, zeros)
    ascf, ascb = _prescale_call(a4, degp)
    out = _mm_call(ascf, ascb, degp)
    return out.reshape(1, NV, NV)


# speedup vs baseline: 1.6674x; 1.0077x over previous
import functools

import numpy as np
import jax
import jax.numpy as jnp
from jax import lax
from jax.experimental import pallas as pl
from jax.experimental.pallas import tpu as pltpu
from jax.experimental.pallas import tpu_sc as plsc

NV = 2048
NE = 32768
Q = NV // 2
QW = Q * Q
NS = 16
NC = 2
EPT = NE // NS
CHUNK_EDGES = 64
CH = 2 * CHUNK_EDGES
NCHUNK = EPT // CHUNK_EDGES
TPW = QW // NS

_N = 33
_n = np.arange(_N, dtype=np.float64)
_x = np.cos(np.pi * (_n + 0.5) / _N)
_kern = np.exp(-2.5 * (_x + 1.0))
_c = np.array([(2.0 / _N) * np.sum(np.cos(np.pi * o * (_n + 0.5) / _N) * _kern)
               for o in range(_N)])
COEF_I = float(0.5 * _c[0] + _c[4::4].sum() - _c[2::4].sum())
COEF_M = float(_c[1] + _c[5::4].sum() - _c[3::4].sum())
COEF_P = float(2.0 * (_c[2::4].sum() + _c[3::4].sum()))
SCALE = float(np.sqrt(NV))


def _sc_scatter_body(ei, ew, zeros, a_out, degp_out,
                     r_v, c_v, w_v, idx_v, didx_v, val0_v, val1_v, dv_v,
                     a_sh, deg_sh, sem_a, sem_d):
    cid = lax.axis_index("c")
    sid = lax.axis_index("s")
    base = sid * EPT

    pltpu.sync_copy(ei.at[0, pl.ds(base, EPT)], r_v)
    pltpu.sync_copy(ei.at[1, pl.ds(base, EPT)], c_v)
    pltpu.sync_copy(ew.at[pl.ds(base, EPT)], w_v)
    pltpu.sync_copy(zeros.at[pl.ds(sid * (NV // NS), NV // NS)],
                    deg_sh.at[pl.ds(sid * (NV // NS), NV // NS)])
    plsc.subcore_barrier()

    def chunk_body(j, carry):
        for v in range(CHUNK_EDGES // 16):
            off = j * CHUNK_EDGES + v * 16
            r16 = r_v[pl.ds(off, 16)]
            c16 = c_v[pl.ds(off, 16)]
            w16 = w_v[pl.ds(off, 16)]
            for d in range(2):
                row = r16 if d == 0 else c16
                col = c16 if d == 0 else r16
                incol = lax.shift_right_logical(col, 10) == cid
                rhalf = lax.shift_right_logical(row, 10)
                hw = jnp.where(incol, w16 * 0.5, 0.0)
                v0 = jnp.where(rhalf == 0, hw, 0.0)
                lidx = lax.shift_left(row & (Q - 1), 10) | (col & (Q - 1))
                ent = d * CHUNK_EDGES + v * 16
                idx_v[j, pl.ds(ent, 16)] = lidx
                didx_v[j, pl.ds(ent, 16)] = row
                val0_v[j, pl.ds(ent, 16)] = v0
                val1_v[j, pl.ds(ent, 16)] = hw - v0
                dv_v[j, pl.ds(ent, 16)] = hw
        pltpu.async_copy(dv_v.at[j], deg_sh.at[didx_v.at[j]], sem_d, add=True)
        return carry

    lax.fori_loop(0, NCHUNK, chunk_body, 0)

    for p, val_v in ((0, val0_v), (1, val1_v)):
        pltpu.sync_copy(zeros, a_sh.at[pl.ds(sid * TPW, TPW)])
        plsc.subcore_barrier()

        def fire_body(j, carry):
            pltpu.async_copy(val_v.at[j], a_sh.at[idx_v.at[j]], sem_a, add=True)
            return carry

        lax.fori_loop(0, NCHUNK, fire_body, 0)

        def drain_body(j, carry):
            pltpu.make_async_copy(val_v.at[j], a_sh.at[idx_v.at[j]], sem_a).wait()
            return carry

        lax.fori_loop(0, NCHUNK, drain_body, 0)
        plsc.subcore_barrier()
        pltpu.sync_copy(a_sh.at[pl.ds(sid * TPW, TPW)], a_out.at[p, cid, sid])
        plsc.subcore_barrier()

    def deg_drain(j, carry):
        pltpu.make_async_copy(dv_v.at[j], deg_sh.at[didx_v.at[j]], sem_d).wait()
        return carry

    lax.fori_loop(0, NCHUNK, deg_drain, 0)
    plsc.subcore_barrier()

    @pl.when(sid == 0)
    def _():
        pltpu.sync_copy(deg_sh, degp_out.at[cid])


def _sc_scatter(ei, ew, zeros):
    mesh = plsc.VectorSubcoreMesh(core_axis_name="c", subcore_axis_name="s")
    f = pl.kernel(
        _sc_scatter_body,
        out_type=[
            jax.ShapeDtypeStruct((2, NC, NS, TPW), jnp.float32),
            jax.ShapeDtypeStruct((NC, NV), jnp.float32),
        ],
        mesh=mesh,
        scratch_types=[
            pltpu.VMEM((EPT,), jnp.int32),
            pltpu.VMEM((EPT,), jnp.int32),
            pltpu.VMEM((EPT,), jnp.float32),
            pltpu.VMEM((NCHUNK, CH), jnp.int32),
            pltpu.VMEM((NCHUNK, CH), jnp.int32),
            pltpu.VMEM((NCHUNK, CH), jnp.float32),
            pltpu.VMEM((NCHUNK, CH), jnp.float32),
            pltpu.VMEM((NCHUNK, CH), jnp.float32),
            pltpu.VMEM_SHARED((QW,), jnp.float32),
            pltpu.VMEM_SHARED((NV,), jnp.float32),
            pltpu.SemaphoreType.DMA,
            pltpu.SemaphoreType.DMA,
        ],
    )
    return f(ei, ew, zeros)


BI = BJ = 512
NI = NV // BI
NJ = NV // BJ


def _prescale_body(a, degp, outf, outb):
    qc = pl.program_id(1)
    degk = degp[0, pl.ds(qc * Q, Q)] + degp[1, pl.ds(qc * Q, Q)]
    dinv2 = jnp.where(degk > 0, 1.0 / degk, 0.0)
    rows = TPW // Q
    for s in range(NS):
        x = a[0, 0, s].reshape(rows, Q) * dinv2[None, :]
        outf[0, 0, pl.ds(s * rows, rows), :] = x
        outb[0, 0, pl.ds(s * rows, rows), :] = x.astype(jnp.bfloat16)


def _prescale_call(a4, degp):
    return pl.pallas_call(
        _prescale_body,
        grid=(2, 2),
        in_specs=[
            pl.BlockSpec((1, 1, NS, TPW), lambda qr, qc: (qr, qc, 0, 0)),
            pl.BlockSpec((NC, NV), lambda qr, qc: (0, 0)),
        ],
        out_specs=[
            pl.BlockSpec((1, 1, Q, Q), lambda qr, qc: (qr, qc, 0, 0)),
            pl.BlockSpec((1, 1, Q, Q), lambda qr, qc: (qr, qc, 0, 0)),
        ],
        out_shape=[
            jax.ShapeDtypeStruct((2, 2, Q, Q), jnp.float32),
            jax.ShapeDtypeStruct((2, 2, Q, Q), jnp.bfloat16),
        ],
        compiler_params=pltpu.CompilerParams(
            dimension_semantics=("parallel", "parallel")),
    )(a4, degp)


def _mm_body(lhs, rhs, aij, degp, out):
    i = pl.program_id(0)
    j = pl.program_id(1)

    acc = (jnp.dot(lhs[0, 0], rhs[0, 0], preferred_element_type=jnp.float32)
           + jnp.dot(lhs[0, 1], rhs[1, 0], preferred_element_type=jnp.float32))

    degi = degp[0, pl.ds(i * BI, BI)] + degp[1, pl.ds(i * BI, BI)]
    degj = degp[0, pl.ds(j * BJ, BJ)] + degp[1, pl.ds(j * BJ, BJ)]
    dinvi = jnp.where(degi > 0, lax.rsqrt(degi), 0.0)
    sdj = jnp.sqrt(degj)
    rows = i * BI + lax.broadcasted_iota(jnp.int32, (BI, BJ), 0)
    cols = j * BJ + lax.broadcasted_iota(jnp.int32, (BI, BJ), 1)
    eye = (rows == cols).astype(jnp.float32)
    dd = dinvi[:, None] * sdj[None, :]
    out[...] = SCALE * (dd * (COEF_P * acc - COEF_M * aij[0, 0])
                        + COEF_I * eye)


def _mm_call(ascf, ascb, degp):
    return pl.pallas_call(
        _mm_body,
        grid=(NI, NJ),
        in_specs=[
            pl.BlockSpec((1, 2, BI, Q), lambda i, j: (i // 2, 0, i % 2, 0)),
            pl.BlockSpec((2, 1, Q, BJ), lambda i, j: (0, j // 2, 0, j % 2)),
            pl.BlockSpec((1, 1, BI, BJ), lambda i, j: (i // 2, j // 2, i % 2, j % 2)),
            pl.BlockSpec((NC, NV), lambda i, j: (0, 0)),
        ],
        out_specs=pl.BlockSpec((BI, BJ), lambda i, j: (i, j)),
        out_shape=jax.ShapeDtypeStruct((NV, NV), jnp.float32),
        compiler_params=pltpu.CompilerParams(
            dimension_semantics=("parallel", "parallel")),
    )(ascb, ascb, ascf, degp)


@jax.jit
def kernel(edge_index, edge_weight):
    zeros = jnp.zeros((TPW,), jnp.float32)
    a4, degp = _sc_scatter(edge_index, edge_weight, zeros)
    ascf, ascb = _prescale_call(a4, degp)
    out = _mm_call(ascf, ascb, degp)
    return out.reshape(1, NV, NV)


# speedup vs baseline: 1.7549x; 1.0525x over previous
import functools

import numpy as np
import jax
import jax.numpy as jnp
from jax import lax
from jax.experimental import pallas as pl
from jax.experimental.pallas import tpu as pltpu
from jax.experimental.pallas import tpu_sc as plsc

NV = 2048
NE = 32768
Q = NV // 2
QW = Q * Q
NS = 16
NC = 2
EPT = NE // NS
CHUNK_EDGES = 64
CH = 2 * CHUNK_EDGES
NCHUNK = EPT // CHUNK_EDGES
TPW = QW // NS

_N = 33
_n = np.arange(_N, dtype=np.float64)
_x = np.cos(np.pi * (_n + 0.5) / _N)
_kern = np.exp(-2.5 * (_x + 1.0))
_c = np.array([(2.0 / _N) * np.sum(np.cos(np.pi * o * (_n + 0.5) / _N) * _kern)
               for o in range(_N)])
COEF_I = float(0.5 * _c[0] + _c[4::4].sum() - _c[2::4].sum())
COEF_M = float(_c[1] + _c[5::4].sum() - _c[3::4].sum())
COEF_P = float(2.0 * (_c[2::4].sum() + _c[3::4].sum()))
SCALE = float(np.sqrt(NV))


def _sc_scatter_body(ei, ew, zeros, a_out, degp_out,
                     r_v, c_v, w_v, idx_v, didx_v, val0_v, val1_v, dv_v,
                     a_sh, deg_sh, sem_a, sem_d):
    cid = lax.axis_index("c")
    sid = lax.axis_index("s")
    base = sid * EPT

    pltpu.sync_copy(ei.at[0, pl.ds(base, EPT)], r_v)
    pltpu.sync_copy(ei.at[1, pl.ds(base, EPT)], c_v)
    pltpu.sync_copy(ew.at[pl.ds(base, EPT)], w_v)
    pltpu.sync_copy(zeros.at[pl.ds(sid * (NV // NS), NV // NS)],
                    deg_sh.at[pl.ds(sid * (NV // NS), NV // NS)])
    plsc.subcore_barrier()

    def chunk_body(j, carry):
        for v in range(CHUNK_EDGES // 16):
            off = j * CHUNK_EDGES + v * 16
            r16 = r_v[pl.ds(off, 16)]
            c16 = c_v[pl.ds(off, 16)]
            w16 = w_v[pl.ds(off, 16)]
            for d in range(2):
                row = r16 if d == 0 else c16
                col = c16 if d == 0 else r16
                incol = lax.shift_right_logical(col, 10) == cid
                rhalf = lax.shift_right_logical(row, 10)
                hw = jnp.where(incol, w16 * 0.5, 0.0)
                v0 = jnp.where(rhalf == 0, hw, 0.0)
                lidx = lax.shift_left(row & (Q - 1), 10) | (col & (Q - 1))
                ent = d * CHUNK_EDGES + v * 16
                idx_v[j, pl.ds(ent, 16)] = lidx
                didx_v[j, pl.ds(ent, 16)] = row
                val0_v[j, pl.ds(ent, 16)] = v0
                val1_v[j, pl.ds(ent, 16)] = hw - v0
                dv_v[j, pl.ds(ent, 16)] = hw
        pltpu.async_copy(dv_v.at[j], deg_sh.at[didx_v.at[j]], sem_d, add=True)
        return carry

    lax.fori_loop(0, NCHUNK, chunk_body, 0)

    for p, val_v in ((0, val0_v), (1, val1_v)):
        pltpu.sync_copy(zeros, a_sh.at[pl.ds(sid * TPW, TPW)])
        plsc.subcore_barrier()

        def fire_body(j, carry):
            pltpu.async_copy(val_v.at[j], a_sh.at[idx_v.at[j]], sem_a, add=True)
            return carry

        lax.fori_loop(0, NCHUNK, fire_body, 0)

        def drain_body(j, carry):
            pltpu.make_async_copy(val_v.at[j], a_sh.at[idx_v.at[j]], sem_a).wait()
            return carry

        lax.fori_loop(0, NCHUNK, drain_body, 0)
        plsc.subcore_barrier()
        pltpu.sync_copy(a_sh.at[pl.ds(sid * TPW, TPW)], a_out.at[p, cid, sid])
        plsc.subcore_barrier()

    def deg_drain(j, carry):
        pltpu.make_async_copy(dv_v.at[j], deg_sh.at[didx_v.at[j]], sem_d).wait()
        return carry

    lax.fori_loop(0, NCHUNK, deg_drain, 0)
    plsc.subcore_barrier()

    @pl.when(sid == 0)
    def _():
        pltpu.sync_copy(deg_sh, degp_out.at[cid])


def _sc_scatter(ei, ew, zeros):
    mesh = plsc.VectorSubcoreMesh(core_axis_name="c", subcore_axis_name="s")
    f = pl.kernel(
        _sc_scatter_body,
        out_type=[
            jax.ShapeDtypeStruct((2, NC, NS, TPW), jnp.float32),
            jax.ShapeDtypeStruct((NC, NV), jnp.float32),
        ],
        mesh=mesh,
        scratch_types=[
            pltpu.VMEM((EPT,), jnp.int32),
            pltpu.VMEM((EPT,), jnp.int32),
            pltpu.VMEM((EPT,), jnp.float32),
            pltpu.VMEM((NCHUNK, CH), jnp.int32),
            pltpu.VMEM((NCHUNK, CH), jnp.int32),
            pltpu.VMEM((NCHUNK, CH), jnp.float32),
            pltpu.VMEM((NCHUNK, CH), jnp.float32),
            pltpu.VMEM((NCHUNK, CH), jnp.float32),
            pltpu.VMEM_SHARED((QW,), jnp.float32),
            pltpu.VMEM_SHARED((NV,), jnp.float32),
            pltpu.SemaphoreType.DMA,
            pltpu.SemaphoreType.DMA,
        ],
    )
    return f(ei, ew, zeros)


BI = BJ = 512
NI = NV // BI
NJ = NV // BJ


def _prescale_body(a, degp, outb):
    qc = pl.program_id(1)
    degk = degp[0, pl.ds(qc * Q, Q)] + degp[1, pl.ds(qc * Q, Q)]
    dinv2 = jnp.where(degk > 0, 1.0 / degk, 0.0)
    rows = TPW // Q
    for s in range(NS):
        x = a[0, 0, s].reshape(rows, Q) * dinv2[None, :]
        outb[0, 0, pl.ds(s * rows, rows), :] = x.astype(jnp.bfloat16)


def _prescale_call(a4, degp):
    return pl.pallas_call(
        _prescale_body,
        grid=(2, 2),
        in_specs=[
            pl.BlockSpec((1, 1, NS, TPW), lambda qr, qc: (qr, qc, 0, 0)),
            pl.BlockSpec((NC, NV), lambda qr, qc: (0, 0)),
        ],
        out_specs=pl.BlockSpec((1, 1, Q, Q), lambda qr, qc: (qr, qc, 0, 0)),
        out_shape=jax.ShapeDtypeStruct((2, 2, Q, Q), jnp.bfloat16),
        compiler_params=pltpu.CompilerParams(
            dimension_semantics=("parallel", "parallel")),
    )(a4, degp)


def _mm_body(lhs, rhs, aij, degp, out):
    i = pl.program_id(0)
    j = pl.program_id(1)

    acc = (jnp.dot(lhs[0, 0], rhs[0, 0], preferred_element_type=jnp.float32)
           + jnp.dot(lhs[0, 1], rhs[1, 0], preferred_element_type=jnp.float32))

    degi = degp[0, pl.ds(i * BI, BI)] + degp[1, pl.ds(i * BI, BI)]
    degj = degp[0, pl.ds(j * BJ, BJ)] + degp[1, pl.ds(j * BJ, BJ)]
    dinvi = jnp.where(degi > 0, lax.rsqrt(degi), 0.0)
    sdj = jnp.sqrt(degj)
    rows = i * BI + lax.broadcasted_iota(jnp.int32, (BI, BJ), 0)
    cols = j * BJ + lax.broadcasted_iota(jnp.int32, (BI, BJ), 1)
    eye = (rows == cols).astype(jnp.float32)
    dd = dinvi[:, None] * sdj[None, :]
    out[...] = SCALE * (dd * (COEF_P * acc
                              - COEF_M * aij[0, 0].astype(jnp.float32))
                        + COEF_I * eye)


def _mm_call(ascb, degp):
    return pl.pallas_call(
        _mm_body,
        grid=(NI, NJ),
        in_specs=[
            pl.BlockSpec((1, 2, BI, Q), lambda i, j: (i // 2, 0, i % 2, 0)),
            pl.BlockSpec((2, 1, Q, BJ), lambda i, j: (0, j // 2, 0, j % 2)),
            pl.BlockSpec((1, 1, BI, BJ), lambda i, j: (i // 2, j // 2, i % 2, j % 2)),
            pl.BlockSpec((NC, NV), lambda i, j: (0, 0)),
        ],
        out_specs=pl.BlockSpec((BI, BJ), lambda i, j: (i, j)),
        out_shape=jax.ShapeDtypeStruct((NV, NV), jnp.float32),
        compiler_params=pltpu.CompilerParams(
            dimension_semantics=("parallel", "parallel")),
    )(ascb, ascb, ascb, degp)


@jax.jit
def kernel(edge_index, edge_weight):
    zeros = jnp.zeros((TPW,), jnp.float32)
    a4, degp = _sc_scatter(edge_index, edge_weight, zeros)
    ascb = _prescale_call(a4, degp)
    out = _mm_call(ascb, degp)
    return out.reshape(1, NV, NV)


# speedup vs baseline: 1.9039x; 1.0849x over previous
import functools

import numpy as np
import jax
import jax.numpy as jnp
from jax import lax
from jax.experimental import pallas as pl
from jax.experimental.pallas import tpu as pltpu
from jax.experimental.pallas import tpu_sc as plsc

NV = 2048
NE = 32768
Q = NV // 2
QW = Q * Q
NS = 16
NC = 2
EPT = NE // NS
CHUNK_EDGES = 64
CH = 2 * CHUNK_EDGES
NCHUNK = EPT // CHUNK_EDGES
TPW = QW // NS

_N = 33
_n = np.arange(_N, dtype=np.float64)
_x = np.cos(np.pi * (_n + 0.5) / _N)
_kern = np.exp(-2.5 * (_x + 1.0))
_c = np.array([(2.0 / _N) * np.sum(np.cos(np.pi * o * (_n + 0.5) / _N) * _kern)
               for o in range(_N)])
COEF_I = float(0.5 * _c[0] + _c[4::4].sum() - _c[2::4].sum())
COEF_M = float(_c[1] + _c[5::4].sum() - _c[3::4].sum())
COEF_P = float(2.0 * (_c[2::4].sum() + _c[3::4].sum()))
SCALE = float(np.sqrt(NV))


def _sc_scatter_body(ei, ew, zeros, a_out, degp_out,
                     r_v, c_v, w_v, idx_v, didx_v, val0_v, val1_v, dv_v,
                     a_sh, deg_sh, sem_a, sem_d):
    cid = lax.axis_index("c")
    sid = lax.axis_index("s")
    base = sid * EPT

    pltpu.sync_copy(ei.at[0, pl.ds(base, EPT)], r_v)
    pltpu.sync_copy(ei.at[1, pl.ds(base, EPT)], c_v)
    pltpu.sync_copy(ew.at[pl.ds(base, EPT)], w_v)
    pltpu.sync_copy(zeros.at[pl.ds(sid * (NV // NS), NV // NS)],
                    deg_sh.at[pl.ds(sid * (NV // NS), NV // NS)])
    plsc.subcore_barrier()

    def chunk_body(j, carry):
        for v in range(CHUNK_EDGES // 16):
            off = j * CHUNK_EDGES + v * 16
            r16 = r_v[pl.ds(off, 16)]
            c16 = c_v[pl.ds(off, 16)]
            w16 = w_v[pl.ds(off, 16)]
            for d in range(2):
                row = r16 if d == 0 else c16
                col = c16 if d == 0 else r16
                incol = lax.shift_right_logical(col, 10) == cid
                rhalf = lax.shift_right_logical(row, 10)
                hw = jnp.where(incol, w16 * 0.5, 0.0)
                v0 = jnp.where(rhalf == 0, hw, 0.0)
                lidx = lax.shift_left(row & (Q - 1), 10) | (col & (Q - 1))
                ent = d * CHUNK_EDGES + v * 16
                idx_v[j, pl.ds(ent, 16)] = lidx
                didx_v[j, pl.ds(ent, 16)] = row
                val0_v[j, pl.ds(ent, 16)] = v0
                val1_v[j, pl.ds(ent, 16)] = hw - v0
                dv_v[j, pl.ds(ent, 16)] = hw
        pltpu.async_copy(dv_v.at[j], deg_sh.at[didx_v.at[j]], sem_d, add=True)
        return carry

    lax.fori_loop(0, NCHUNK, chunk_body, 0)

    for p, val_v in ((0, val0_v), (1, val1_v)):
        pltpu.sync_copy(zeros, a_sh.at[pl.ds(sid * TPW, TPW)])
        plsc.subcore_barrier()

        def fire_body(j, carry):
            pltpu.async_copy(val_v.at[j], a_sh.at[idx_v.at[j]], sem_a, add=True)
            return carry

        lax.fori_loop(0, NCHUNK, fire_body, 0)

        def drain_body(j, carry):
            pltpu.make_async_copy(val_v.at[j], a_sh.at[idx_v.at[j]], sem_a).wait()
            return carry

        lax.fori_loop(0, NCHUNK, drain_body, 0)
        plsc.subcore_barrier()
        pltpu.sync_copy(a_sh.at[pl.ds(sid * TPW, TPW)], a_out.at[p, cid, sid])
        plsc.subcore_barrier()

    def deg_drain(j, carry):
        pltpu.make_async_copy(dv_v.at[j], deg_sh.at[didx_v.at[j]], sem_d).wait()
        return carry

    lax.fori_loop(0, NCHUNK, deg_drain, 0)
    plsc.subcore_barrier()

    @pl.when(sid == 0)
    def _():
        pltpu.sync_copy(deg_sh, degp_out.at[cid])


def _sc_scatter(ei, ew, zeros):
    mesh = plsc.VectorSubcoreMesh(core_axis_name="c", subcore_axis_name="s")
    f = pl.kernel(
        _sc_scatter_body,
        out_type=[
            jax.ShapeDtypeStruct((2, NC, NS, TPW), jnp.float32),
            jax.ShapeDtypeStruct((NC, NV), jnp.float32),
        ],
        mesh=mesh,
        scratch_types=[
            pltpu.VMEM((EPT,), jnp.int32),
            pltpu.VMEM((EPT,), jnp.int32),
            pltpu.VMEM((EPT,), jnp.float32),
            pltpu.VMEM((NCHUNK, CH), jnp.int32),
            pltpu.VMEM((NCHUNK, CH), jnp.int32),
            pltpu.VMEM((NCHUNK, CH), jnp.float32),
            pltpu.VMEM((NCHUNK, CH), jnp.float32),
            pltpu.VMEM((NCHUNK, CH), jnp.float32),
            pltpu.VMEM_SHARED((QW,), jnp.float32),
            pltpu.VMEM_SHARED((NV,), jnp.float32),
            pltpu.SemaphoreType.DMA,
            pltpu.SemaphoreType.DMA,
        ],
    )
    return f(ei, ew, zeros)


BI = BJ = 1024
NI = NV // BI
NJ = NV // BJ
BPQ = max(Q // BI, 1)


def _prescale_body(a, degp, outb):
    qc = pl.program_id(1)
    degk = degp[0, pl.ds(qc * Q, Q)] + degp[1, pl.ds(qc * Q, Q)]
    dinv2 = jnp.where(degk > 0, 1.0 / degk, 0.0)
    rows = TPW // Q
    for s in range(NS):
        x = a[0, 0, s].reshape(rows, Q) * dinv2[None, :]
        outb[0, 0, pl.ds(s * rows, rows), :] = x.astype(jnp.bfloat16)


def _prescale_call(a4, degp):
    return pl.pallas_call(
        _prescale_body,
        grid=(2, 2),
        in_specs=[
            pl.BlockSpec((1, 1, NS, TPW), lambda qr, qc: (qr, qc, 0, 0)),
            pl.BlockSpec((NC, NV), lambda qr, qc: (0, 0)),
        ],
        out_specs=pl.BlockSpec((1, 1, Q, Q), lambda qr, qc: (qr, qc, 0, 0)),
        out_shape=jax.ShapeDtypeStruct((2, 2, Q, Q), jnp.bfloat16),
        compiler_params=pltpu.CompilerParams(
            dimension_semantics=("parallel", "parallel")),
    )(a4, degp)


def _mm_body(lhs, rhs, aij, degp, out):
    i = pl.program_id(0)
    j = pl.program_id(1)

    acc = (jnp.dot(lhs[0, 0], rhs[0, 0], preferred_element_type=jnp.float32)
           + jnp.dot(lhs[0, 1], rhs[1, 0], preferred_element_type=jnp.float32))

    degi = degp[0, pl.ds(i * BI, BI)] + degp[1, pl.ds(i * BI, BI)]
    degj = degp[0, pl.ds(j * BJ, BJ)] + degp[1, pl.ds(j * BJ, BJ)]
    dinvi = jnp.where(degi > 0, lax.rsqrt(degi), 0.0)
    sdj = jnp.sqrt(degj)
    rows = i * BI + lax.broadcasted_iota(jnp.int32, (BI, BJ), 0)
    cols = j * BJ + lax.broadcasted_iota(jnp.int32, (BI, BJ), 1)
    eye = (rows == cols).astype(jnp.float32)
    dd = dinvi[:, None] * sdj[None, :]
    out[...] = SCALE * (dd * (COEF_P * acc
                              - COEF_M * aij[0, 0].astype(jnp.float32))
                        + COEF_I * eye)


def _mm_call(ascb, degp):
    return pl.pallas_call(
        _mm_body,
        grid=(NI, NJ),
        in_specs=[
            pl.BlockSpec((1, 2, BI, Q), lambda i, j: (i // BPQ, 0, i % BPQ, 0)),
            pl.BlockSpec((2, 1, Q, BJ), lambda i, j: (0, j // BPQ, 0, j % BPQ)),
            pl.BlockSpec((1, 1, BI, BJ),
                         lambda i, j: (i // BPQ, j // BPQ, i % BPQ, j % BPQ)),
            pl.BlockSpec((NC, NV), lambda i, j: (0, 0)),
        ],
        out_specs=pl.BlockSpec((BI, BJ), lambda i, j: (i, j)),
        out_shape=jax.ShapeDtypeStruct((NV, NV), jnp.float32),
        compiler_params=pltpu.CompilerParams(
            dimension_semantics=("parallel", "parallel")),
    )(ascb, ascb, ascb, degp)


@jax.jit
def kernel(edge_index, edge_weight):
    zeros = jnp.zeros((TPW,), jnp.float32)
    a4, degp = _sc_scatter(edge_index, edge_weight, zeros)
    ascb = _prescale_call(a4, degp)
    out = _mm_call(ascb, degp)
    return out.reshape(1, NV, NV)


# speedup vs baseline: 1.9471x; 1.0227x over previous
import functools

import numpy as np
import jax
import jax.numpy as jnp
from jax import lax
from jax.experimental import pallas as pl
from jax.experimental.pallas import tpu as pltpu
from jax.experimental.pallas import tpu_sc as plsc

NV = 2048
NE = 32768
Q = NV // 2
QW = Q * Q
NS = 16
NC = 2
EPT = NE // NS
CHUNK_EDGES = 64
CH = 2 * CHUNK_EDGES
NCHUNK = EPT // CHUNK_EDGES
TPW = QW // NS

_N = 33
_n = np.arange(_N, dtype=np.float64)
_x = np.cos(np.pi * (_n + 0.5) / _N)
_kern = np.exp(-2.5 * (_x + 1.0))
_c = np.array([(2.0 / _N) * np.sum(np.cos(np.pi * o * (_n + 0.5) / _N) * _kern)
               for o in range(_N)])
COEF_I = float(0.5 * _c[0] + _c[4::4].sum() - _c[2::4].sum())
COEF_M = float(_c[1] + _c[5::4].sum() - _c[3::4].sum())
COEF_P = float(2.0 * (_c[2::4].sum() + _c[3::4].sum()))
SCALE = float(np.sqrt(NV))


def _sc_scatter_body(ei, ew, zeros, a_out, degp_out,
                     r_v, c_v, w_v, idx_v, didx_v, val0_v, val1_v, dv_v,
                     a_sh, deg_sh, sem_a, sem_d, sem_z):
    cid = lax.axis_index("c")
    sid = lax.axis_index("s")
    base = sid * EPT

    pltpu.sync_copy(ei.at[0, pl.ds(base, EPT)], r_v)
    pltpu.sync_copy(ei.at[1, pl.ds(base, EPT)], c_v)
    pltpu.sync_copy(ew.at[pl.ds(base, EPT)], w_v)
    pltpu.sync_copy(zeros.at[pl.ds(sid * (NV // NS), NV // NS)],
                    deg_sh.at[pl.ds(sid * (NV // NS), NV // NS)])
    zcopy = pltpu.make_async_copy(zeros, a_sh.at[pl.ds(sid * TPW, TPW)], sem_z)
    zcopy.start()
    plsc.subcore_barrier()

    def chunk_body(j, carry):
        for v in range(CHUNK_EDGES // 16):
            off = j * CHUNK_EDGES + v * 16
            r16 = r_v[pl.ds(off, 16)]
            c16 = c_v[pl.ds(off, 16)]
            w16 = w_v[pl.ds(off, 16)]
            for d in range(2):
                row = r16 if d == 0 else c16
                col = c16 if d == 0 else r16
                incol = lax.shift_right_logical(col, 10) == cid
                rhalf = lax.shift_right_logical(row, 10)
                hw = jnp.where(incol, w16 * 0.5, 0.0)
                v0 = jnp.where(rhalf == 0, hw, 0.0)
                lidx = lax.shift_left(row & (Q - 1), 10) | (col & (Q - 1))
                ent = d * CHUNK_EDGES + v * 16
                idx_v[j, pl.ds(ent, 16)] = lidx
                didx_v[j, pl.ds(ent, 16)] = row
                val0_v[j, pl.ds(ent, 16)] = v0
                val1_v[j, pl.ds(ent, 16)] = hw - v0
                dv_v[j, pl.ds(ent, 16)] = hw
        pltpu.async_copy(dv_v.at[j], deg_sh.at[didx_v.at[j]], sem_d, add=True)
        return carry

    lax.fori_loop(0, NCHUNK, chunk_body, 0)

    for p, val_v in ((0, val0_v), (1, val1_v)):
        if p == 0:
            zcopy.wait()
        else:
            pltpu.sync_copy(zeros, a_sh.at[pl.ds(sid * TPW, TPW)])
        plsc.subcore_barrier()

        def fire_body(j, carry):
            pltpu.async_copy(val_v.at[j], a_sh.at[idx_v.at[j]], sem_a, add=True)
            return carry

        lax.fori_loop(0, NCHUNK, fire_body, 0)

        def drain_body(j, carry):
            pltpu.make_async_copy(val_v.at[j], a_sh.at[idx_v.at[j]], sem_a).wait()
            return carry

        lax.fori_loop(0, NCHUNK, drain_body, 0)
        plsc.subcore_barrier()
        pltpu.sync_copy(a_sh.at[pl.ds(sid * TPW, TPW)], a_out.at[p, cid, sid])
        plsc.subcore_barrier()

    def deg_drain(j, carry):
        pltpu.make_async_copy(dv_v.at[j], deg_sh.at[didx_v.at[j]], sem_d).wait()
        return carry

    lax.fori_loop(0, NCHUNK, deg_drain, 0)
    plsc.subcore_barrier()

    @pl.when(sid == 0)
    def _():
        pltpu.sync_copy(deg_sh, degp_out.at[cid])


def _sc_scatter(ei, ew, zeros):
    mesh = plsc.VectorSubcoreMesh(core_axis_name="c", subcore_axis_name="s")
    f = pl.kernel(
        _sc_scatter_body,
        out_type=[
            jax.ShapeDtypeStruct((2, NC, NS, TPW), jnp.float32),
            jax.ShapeDtypeStruct((NC, NV), jnp.float32),
        ],
        mesh=mesh,
        scratch_types=[
            pltpu.VMEM((EPT,), jnp.int32),
            pltpu.VMEM((EPT,), jnp.int32),
            pltpu.VMEM((EPT,), jnp.float32),
            pltpu.VMEM((NCHUNK, CH), jnp.int32),
            pltpu.VMEM((NCHUNK, CH), jnp.int32),
            pltpu.VMEM((NCHUNK, CH), jnp.float32),
            pltpu.VMEM((NCHUNK, CH), jnp.float32),
            pltpu.VMEM((NCHUNK, CH), jnp.float32),
            pltpu.VMEM_SHARED((QW,), jnp.float32),
            pltpu.VMEM_SHARED((NV,), jnp.float32),
            pltpu.SemaphoreType.DMA,
            pltpu.SemaphoreType.DMA,
            pltpu.SemaphoreType.DMA,
        ],
    )
    return f(ei, ew, zeros)


BI = BJ = 1024
NI = NV // BI
NJ = NV // BJ
BPQ = max(Q // BI, 1)


def _prescale_body(a, degp, outb):
    qc = pl.program_id(1)
    degk = degp[0, pl.ds(qc * Q, Q)] + degp[1, pl.ds(qc * Q, Q)]
    dinv2 = jnp.where(degk > 0, 1.0 / degk, 0.0)
    rows = TPW // Q
    for s in range(NS):
        x = a[0, 0, s].reshape(rows, Q) * dinv2[None, :]
        outb[0, 0, pl.ds(s * rows, rows), :] = x.astype(jnp.bfloat16)


def _prescale_call(a4, degp):
    return pl.pallas_call(
        _prescale_body,
        grid=(2, 2),
        in_specs=[
            pl.BlockSpec((1, 1, NS, TPW), lambda qr, qc: (qr, qc, 0, 0)),
            pl.BlockSpec((NC, NV), lambda qr, qc: (0, 0)),
        ],
        out_specs=pl.BlockSpec((1, 1, Q, Q), lambda qr, qc: (qr, qc, 0, 0)),
        out_shape=jax.ShapeDtypeStruct((2, 2, Q, Q), jnp.bfloat16),
        compiler_params=pltpu.CompilerParams(
            dimension_semantics=("parallel", "parallel")),
    )(a4, degp)


def _mm_body(lhs, rhs, aij, degp, out):
    i = pl.program_id(0)
    j = pl.program_id(1)

    acc = (jnp.dot(lhs[0, 0], rhs[0, 0], preferred_element_type=jnp.float32)
           + jnp.dot(lhs[0, 1], rhs[1, 0], preferred_element_type=jnp.float32))

    degi = degp[0, pl.ds(i * BI, BI)] + degp[1, pl.ds(i * BI, BI)]
    degj = degp[0, pl.ds(j * BJ, BJ)] + degp[1, pl.ds(j * BJ, BJ)]
    dinvi = jnp.where(degi > 0, lax.rsqrt(degi), 0.0)
    sdj = jnp.sqrt(degj)
    rows = i * BI + lax.broadcasted_iota(jnp.int32, (BI, BJ), 0)
    cols = j * BJ + lax.broadcasted_iota(jnp.int32, (BI, BJ), 1)
    eye = (rows == cols).astype(jnp.float32)
    dd = dinvi[:, None] * sdj[None, :]
    out[...] = SCALE * (dd * (COEF_P * acc
                              - COEF_M * aij[0, 0].astype(jnp.float32))
                        + COEF_I * eye)


def _mm_call(ascb, degp):
    return pl.pallas_call(
        _mm_body,
        grid=(NI, NJ),
        in_specs=[
            pl.BlockSpec((1, 2, BI, Q), lambda i, j: (i // BPQ, 0, i % BPQ, 0)),
            pl.BlockSpec((2, 1, Q, BJ), lambda i, j: (0, j // BPQ, 0, j % BPQ)),
            pl.BlockSpec((1, 1, BI, BJ),
                         lambda i, j: (i // BPQ, j // BPQ, i % BPQ, j % BPQ)),
            pl.BlockSpec((NC, NV), lambda i, j: (0, 0)),
        ],
        out_specs=pl.BlockSpec((BI, BJ), lambda i, j: (i, j)),
        out_shape=jax.ShapeDtypeStruct((NV, NV), jnp.float32),
        compiler_params=pltpu.CompilerParams(
            dimension_semantics=("parallel", "parallel")),
    )(ascb, ascb, ascb, degp)


@jax.jit
def kernel(edge_index, edge_weight):
    zeros = jnp.zeros((TPW,), jnp.float32)
    a4, degp = _sc_scatter(edge_index, edge_weight, zeros)
    ascb = _prescale_call(a4, degp)
    out = _mm_call(ascb, degp)
    return out.reshape(1, NV, NV)
